# baseline (speedup 1.0000x reference)
import functools

import jax
import jax.numpy as jnp
from jax import lax
from jax.experimental import pallas as pl
from jax.experimental.pallas import tpu as pltpu
from jax.experimental.pallas import tpu_sc as plsc

B, V, D, H = 16384, 100000, 64, 512

WIN = 768
NWIN = (V + WIN - 1) // WIN
TAIL_START = (NWIN - 1) * WIN
TAIL_LEN = V - TAIL_START
LANE_CAP = 512
E_ROWS = B + 512


def _make_sc_stream_gather():
    info = plsc.get_sparse_core_info()
    NC, NS = info.num_cores, info.num_subcores
    half = B // NC
    n_j = (NWIN + NS - 1) // NS

    mesh = plsc.VectorSubcoreMesh(core_axis_name="c", subcore_axis_name="s")

    @functools.partial(
        pl.kernel,
        mesh=mesh,
        out_type=jax.ShapeDtypeStruct((E_ROWS, 128), jnp.float32),
        scratch_types=[
            pltpu.VMEM((D, WIN), jnp.float32),
            pltpu.VMEM((D, TAIL_LEN), jnp.float32),
            pltpu.VMEM((half,), jnp.int32),
            pltpu.VMEM((16 * LANE_CAP + 16,), jnp.int32),
            pltpu.VMEM((16 * LANE_CAP + 16,), jnp.int32),
            pltpu.VMEM((128, 128), jnp.float32),
            pltpu.VMEM((128,), jnp.int32),
            pltpu.SemaphoreType.DMA,
        ],
        compiler_params=pltpu.CompilerParams(
            use_tc_tiling_on_sc=True, needs_layout_passes=False
        ),
    )
    def gather_kernel(idx_hbm, tt_hbm, out_hbm, buf, tailbuf, idxv, wvstore,
                      bstore, stg, bidx, sem):
        c = lax.axis_index("c")
        s = lax.axis_index("s")
        wid = s * NC + c
        lanes = lax.iota(jnp.int32, 16)
        zeros16 = jnp.zeros((16,), jnp.float32)
        dump_rows = B + wid * 16 + lanes
        base_b = c * half

        pltpu.sync_copy(idx_hbm.at[pl.ds(base_b, half)], idxv)

        def zero_body(r, carry):
            for cc in range(D, 128, 16):
                plsc.store_scatter(
                    stg, [jnp.full((16,), r, jnp.int32), cc + lanes], zeros16)
            return carry
        lax.fori_loop(0, 128, zero_body, 0)

        def j_body(j, carry):
            w = j * NS + s

            @pl.when(w < NWIN)
            def _():
                wstart = w * WIN

                @pl.when(w < NWIN - 1)
                def _():
                    pltpu.sync_copy(tt_hbm.at[:, pl.ds(wstart, WIN)], buf)

                @pl.when(w == NWIN - 1)
                def _():
                    pltpu.sync_copy(
                        tt_hbm.at[:, pl.ds(TAIL_START, TAIL_LEN)], tailbuf)

                def g_body(g, cnt_vec):
                    v = idxv[pl.ds(g * 16, 16)]
                    wv = v - wstart
                    m = (wv >= 0) & (wv < WIN)
                    offs = lanes * LANE_CAP + cnt_vec
                    plsc.store_scatter(wvstore, [offs], wv, mask=m)
                    plsc.store_scatter(
                        bstore, [offs], base_b + g * 16 + lanes, mask=m)
                    return cnt_vec + m.astype(jnp.int32)

                cnt_vec = lax.fori_loop(
                    0, half // 16, g_body, jnp.zeros((16,), jnp.int32))
                kmax = jnp.max(cnt_vec)

                def make_q_body(src):
                    def q_body(q, carry):
                        for u in range(8):
                            k = q * 8 + u
                            m_k = k < cnt_vec
                            wvk = plsc.load_gather(
                                wvstore, [lanes * LANE_CAP + k])
                            bvk = plsc.load_gather(
                                bstore, [lanes * LANE_CAP + k])
                            wv_safe = jnp.where(m_k, wvk, 0)
                            b_safe = jnp.where(m_k, bvk, dump_rows)
                            bidx[pl.ds(u * 16, 16)] = b_safe
                            rows = u * 16 + lanes
                            for d in range(D):
                                vals = plsc.load_gather(
                                    src,
                                    [jnp.full((16,), d, jnp.int32), wv_safe])
                                plsc.store_scatter(
                                    stg, [rows, jnp.full((16,), d, jnp.int32)],
                                    vals)
                        pltpu.async_copy(stg, out_hbm.at[bidx], sem).wait()
                        return carry
                    return q_body

                nq = (kmax + 7) // 8

                @pl.when(w < NWIN - 1)
                def _():
                    lax.fori_loop(0, nq, make_q_body(buf), 0)

                @pl.when(w == NWIN - 1)
                def _():
                    lax.fori_loop(0, nq, make_q_body(tailbuf), 0)

            return carry

        lax.fori_loop(0, n_j, j_body, 0)

    return gather_kernel


_sc_gather = _make_sc_stream_gather()


BK = 1024


def _mlp_body(e_ref, W1_ref, b1_ref, W2_ref, b2_ref, W3_ref, b3_ref, out_ref):
    e = e_ref[...]
    h = jnp.tanh(
        jax.lax.dot_general(e, W1_ref[...], (((1,), (0,)), ((), ())),
                            preferred_element_type=jnp.float32)
        + b1_ref[...])
    h = jnp.tanh(
        jax.lax.dot_general(h, W2_ref[...], (((1,), (0,)), ((), ())),
                            preferred_element_type=jnp.float32)
        + b2_ref[...])
    out_ref[...] = (
        jax.lax.dot_general(h, W3_ref[...], (((1,), (0,)), ((), ())),
                            preferred_element_type=jnp.float32)
        + b3_ref[...])


def _mlp(e, W1p, b1, W2, b2, W3, b3):
    grid = (B // BK,)
    return pl.pallas_call(
        _mlp_body,
        grid=grid,
        in_specs=[
            pl.BlockSpec((BK, 128), lambda i: (i, 0)),
            pl.BlockSpec((128, H), lambda i: (0, 0)),
            pl.BlockSpec((1, H), lambda i: (0, 0)),
            pl.BlockSpec((H, H), lambda i: (0, 0)),
            pl.BlockSpec((1, H), lambda i: (0, 0)),
            pl.BlockSpec((H, 1), lambda i: (0, 0)),
            pl.BlockSpec((1, 1), lambda i: (0, 0)),
        ],
        out_specs=pl.BlockSpec((BK, 1), lambda i: (i, 0)),
        out_shape=jax.ShapeDtypeStruct((B, 1), jnp.float32),
    )(e, W1p, b1, W2, b2, W3, b3)


def kernel(x, table, W1, b1, W2, b2, W3, b3):
    idx = jnp.reshape(x, (B,)).astype(jnp.int32)
    e = _sc_gather(idx, table.T)
    W1p = jnp.pad(W1, ((0, 128 - D), (0, 0)))
    return _mlp(e, W1p, b1.reshape(1, H), W2, b2.reshape(1, H),
                W3, b3.reshape(1, 1))


# speedup vs baseline: 1.3050x; 1.3050x over previous
import functools

import jax
import jax.numpy as jnp
from jax import lax
from jax.experimental import pallas as pl
from jax.experimental.pallas import tpu as pltpu
from jax.experimental.pallas import tpu_sc as plsc

B, V, D, H = 16384, 100000, 64, 512


def _make_sc_gather():
    info = plsc.get_sparse_core_info()
    NC, NS = info.num_cores, info.num_subcores
    NW = NC * NS
    b_per_w = B // NW
    mesh = plsc.VectorSubcoreMesh(core_axis_name="c", subcore_axis_name="s")

    @functools.partial(
        pl.kernel,
        mesh=mesh,
        out_type=jax.ShapeDtypeStruct((B, D), jnp.bfloat16),
        scratch_types=[
            pltpu.VMEM((b_per_w,), jnp.int32),
            pltpu.VMEM((b_per_w, D), jnp.bfloat16),
            pltpu.SemaphoreType.DMA,
        ],
        compiler_params=pltpu.CompilerParams(use_tc_tiling_on_sc=False),
    )
    def gather_kernel(idx_hbm, table_hbm, out_hbm, idx_v, rows_v, sem):
        wid = lax.axis_index("s") * NC + lax.axis_index("c")
        base = wid * b_per_w
        pltpu.sync_copy(idx_hbm.at[pl.ds(base, b_per_w)], idx_v)
        pltpu.async_copy(table_hbm.at[idx_v], rows_v, sem).wait()
        pltpu.sync_copy(rows_v, out_hbm.at[pl.ds(base, b_per_w)])

    return gather_kernel


_sc_gather = _make_sc_gather()


BK = 1024


def _mlp_body(e_ref, W1_ref, b1_ref, W2_ref, b2_ref, W3_ref, b3_ref, out_ref):
    h = jnp.tanh(
        jax.lax.dot_general(e_ref[...], W1_ref[...], (((1,), (0,)), ((), ())),
                            preferred_element_type=jnp.float32)
        + b1_ref[...])
    h = jnp.tanh(
        jax.lax.dot_general(h.astype(jnp.bfloat16), W2_ref[...],
                            (((1,), (0,)), ((), ())),
                            preferred_element_type=jnp.float32)
        + b2_ref[...])
    out_ref[...] = (
        jax.lax.dot_general(h, W3_ref[...], (((1,), (0,)), ((), ())),
                            preferred_element_type=jnp.float32)
        + b3_ref[...])


def _mlp(e, W1, b1, W2, b2, W3, b3):
    grid = (B // BK,)
    return pl.pallas_call(
        _mlp_body,
        grid=grid,
        in_specs=[
            pl.BlockSpec((BK, D), lambda i: (i, 0)),
            pl.BlockSpec((D, H), lambda i: (0, 0)),
            pl.BlockSpec((1, H), lambda i: (0, 0)),
            pl.BlockSpec((H, H), lambda i: (0, 0)),
            pl.BlockSpec((1, H), lambda i: (0, 0)),
            pl.BlockSpec((H, 1), lambda i: (0, 0)),
            pl.BlockSpec((1, 1), lambda i: (0, 0)),
        ],
        out_specs=pl.BlockSpec((BK, 1), lambda i: (i, 0)),
        out_shape=jax.ShapeDtypeStruct((B, 1), jnp.float32),
    )(e, W1, b1, W2, b2, W3, b3)


def kernel(x, table, W1, b1, W2, b2, W3, b3):
    idx = jnp.reshape(x, (B,)).astype(jnp.int32)
    e = _sc_gather(idx, table.astype(jnp.bfloat16))
    return _mlp(e, W1.astype(jnp.bfloat16), b1.reshape(1, H),
                W2.astype(jnp.bfloat16), b2.reshape(1, H),
                W3, b3.reshape(1, 1))


# speedup vs baseline: 2.3529x; 1.8029x over previous
import functools

import jax
import jax.numpy as jnp
from jax import lax
from jax.experimental import pallas as pl
from jax.experimental.pallas import tpu as pltpu
from jax.experimental.pallas import tpu_sc as plsc

B, V, D, H = 16384, 100000, 64, 512


TBV = 6400
NTB = (V + TBV - 1) // TBV


def _tpose_body(tt_ref, out_ref):
    xt = tt_ref[...].T
    out_ref[...] = jnp.concatenate([xt, jnp.zeros_like(xt)], axis=-1)


def _tpose(tt):
    return pl.pallas_call(
        _tpose_body,
        grid=(NTB,),
        in_specs=[pl.BlockSpec((D, TBV), lambda i: (0, i))],
        out_specs=pl.BlockSpec((TBV, 128), lambda i: (i, 0)),
        out_shape=jax.ShapeDtypeStruct((V, 128), jnp.float32),
    )(tt)


def _make_sc_gather():
    info = plsc.get_sparse_core_info()
    NC, NS = info.num_cores, info.num_subcores
    NW = NC * NS
    b_per_w = B // NW
    mesh = plsc.VectorSubcoreMesh(core_axis_name="c", subcore_axis_name="s")

    @functools.partial(
        pl.kernel,
        mesh=mesh,
        out_type=jax.ShapeDtypeStruct((B, 128), jnp.float32),
        scratch_types=[
            pltpu.VMEM((b_per_w,), jnp.int32),
            pltpu.VMEM((b_per_w, 128), jnp.float32),
            pltpu.SemaphoreType.DMA,
        ],
        compiler_params=pltpu.CompilerParams(use_tc_tiling_on_sc=False),
    )
    def gather_kernel(idx_hbm, tp_hbm, out_hbm, idx_v, rows_v, sem):
        wid = lax.axis_index("s") * NC + lax.axis_index("c")
        base = wid * b_per_w
        pltpu.sync_copy(idx_hbm.at[pl.ds(base, b_per_w)], idx_v)
        pltpu.async_copy(tp_hbm.at[idx_v], rows_v, sem).wait()
        pltpu.sync_copy(rows_v, out_hbm.at[pl.ds(base, b_per_w)])

    return gather_kernel


_sc_gather = _make_sc_gather()


BK = 1024


def _mlp_body(e_ref, W1_ref, b1_ref, W2_ref, b2_ref, W3_ref, b3_ref, out_ref):
    h = jnp.tanh(
        jax.lax.dot_general(e_ref[...], W1_ref[...], (((1,), (0,)), ((), ())),
                            preferred_element_type=jnp.float32)
        + b1_ref[...])
    h = jnp.tanh(
        jax.lax.dot_general(h, W2_ref[...], (((1,), (0,)), ((), ())),
                            preferred_element_type=jnp.float32)
        + b2_ref[...])
    out_ref[...] = (
        jax.lax.dot_general(h, W3_ref[...], (((1,), (0,)), ((), ())),
                            preferred_element_type=jnp.float32)
        + b3_ref[...])


def _mlp(e, W1p, b1, W2, b2, W3, b3):
    grid = (B // BK,)
    return pl.pallas_call(
        _mlp_body,
        grid=grid,
        in_specs=[
            pl.BlockSpec((BK, 128), lambda i: (i, 0)),
            pl.BlockSpec((128, H), lambda i: (0, 0)),
            pl.BlockSpec((1, H), lambda i: (0, 0)),
            pl.BlockSpec((H, H), lambda i: (0, 0)),
            pl.BlockSpec((1, H), lambda i: (0, 0)),
            pl.BlockSpec((H, 1), lambda i: (0, 0)),
            pl.BlockSpec((1, 1), lambda i: (0, 0)),
        ],
        out_specs=pl.BlockSpec((BK, 1), lambda i: (i, 0)),
        out_shape=jax.ShapeDtypeStruct((B, 1), jnp.float32),
    )(e, W1p, b1, W2, b2, W3, b3)


def kernel(x, table, W1, b1, W2, b2, W3, b3):
    idx = jnp.reshape(x, (B,)).astype(jnp.int32)
    tp = _tpose(table.T)
    e = _sc_gather(idx, tp)
    W1p = jnp.pad(W1, ((0, 128 - D), (0, 0)))
    return _mlp(e, W1p, b1.reshape(1, H), W2, b2.reshape(1, H),
                W3, b3.reshape(1, 1))
